# Initial kernel scaffold; baseline (speedup 1.0000x reference)
import jax
import jax.numpy as jnp
from jax.experimental import pallas as pl


def kernel(x, params):
    raise NotImplementedError("write your pallas kernel here")


# speedup vs baseline: 7.1387x; 7.1387x over previous
import functools
import math

import jax
import jax.numpy as jnp
from jax import lax
from jax.experimental import pallas as pl
from jax.experimental.pallas import tpu as pltpu

_EPS = 1e-5
_K = 8
_CH = [(3, 32), (32, 64), (64, 128), (128, 256), (256, 512)]
_BNS = float(1.0 / math.sqrt(1.0 + _EPS))


def _softmax_rows(a):
    m = jnp.max(a, axis=1, keepdims=True)
    e = jnp.exp(a - m)
    return e / jnp.sum(e, axis=1, keepdims=True)


def _layer_kernel(xb_ref, q_ref, wxt_ref, bxt_ref, gxt_ref, bexr_ref,
                  wc_ref, bc_ref, gc_ref, bec_ref, out_ref):
    xb = xb_ref[0]
    q = q_ref[0]
    R = q.shape[0]
    N = xb.shape[0]

    qn = jnp.sum(q * q, axis=1, keepdims=True)
    xn = jnp.sum(xb * xb, axis=1)[None, :]
    d = qn + xn - 2.0 * lax.dot_general(
        q, xb, (((1,), (1,)), ((), ())),
        preferred_element_type=jnp.float32)

    iota = lax.broadcasted_iota(jnp.int32, (R, N), 1)
    x0 = xb[:, 0][None, :]

    xg = []
    xtcols = []
    for _ in range(_K):
        m = jnp.min(d, axis=1, keepdims=True)
        ismin = d == m
        sel = jnp.min(jnp.where(ismin, iota, N), axis=1, keepdims=True)
        oh = iota == sel
        ohf = oh.astype(jnp.float32)
        xtcols.append(jnp.sum(ohf * x0, axis=1, keepdims=True))
        xg.append(lax.dot_general(ohf, xb, (((1,), (0,)), ((), ())),
                                  preferred_element_type=jnp.float32))
        d = jnp.where(oh, jnp.float32(1e30), d)

    xt = jnp.concatenate(xtcols, axis=1)
    X = lax.dot_general(xt, wxt_ref[...], (((1,), (1,)), ((), ())),
                        preferred_element_type=jnp.float32)
    X = X + bxt_ref[...][None, :]
    X = X * _BNS * gxt_ref[...][None, :] + bexr_ref[...][None, :]

    g = None
    for i in range(_K):
        s = _softmax_rows(X[:, i * _K:(i + 1) * _K])
        gt = s[:, 0:1] * xg[0]
        for j in range(1, _K):
            gt = gt + s[:, j:j + 1] * xg[j]
        g = gt if g is None else jnp.maximum(g, gt)

    out = lax.dot_general(g, wc_ref[...], (((1,), (1,)), ((), ())),
                          preferred_element_type=jnp.float32)
    out = out + bc_ref[...][None, :]
    out = out * _BNS * gc_ref[...][None, :] + bec_ref[...][None, :]
    out_ref[0] = jnp.maximum(out, 0.0)


def _xconv_layer(x, p, c_out, block_r=256):
    B, N, C = x.shape
    grid = (B, N // block_r)
    full = lambda shape: pl.BlockSpec(shape, lambda b, r: (0,) * len(shape))
    return pl.pallas_call(
        _layer_kernel,
        grid=grid,
        in_specs=[
            pl.BlockSpec((1, N, C), lambda b, r: (b, 0, 0)),
            pl.BlockSpec((1, block_r, C), lambda b, r: (b, r, 0)),
            full((_K * _K, _K)),
            full((_K * _K,)),
            full((_K * _K,)),
            full((_K * _K,)),
            full((c_out, C)),
            full((c_out,)),
            full((c_out,)),
            full((c_out,)),
        ],
        out_specs=pl.BlockSpec((1, block_r, c_out), lambda b, r: (b, r, 0)),
        out_shape=jax.ShapeDtypeStruct((B, N, c_out), jnp.float32),
    )(x, x, p['Wxt'], p['bxt'], p['gxt'], p['betaxt'],
      p['Wc'], p['bc'], p['gc'], p['betac'])


def _head_kernel(h_ref, w1_ref, b1_ref, w2_ref, b2_ref, out_ref):
    h = h_ref[0]
    m = jnp.max(h, axis=0, keepdims=True)
    f = lax.dot_general(m, w1_ref[...], (((1,), (1,)), ((), ())),
                        preferred_element_type=jnp.float32)
    f = jnp.maximum(f + b1_ref[...][None, :], 0.0)
    o = lax.dot_general(f, w2_ref[...], (((1,), (1,)), ((), ())),
                        preferred_element_type=jnp.float32)
    out_ref[0, 0] = o[0] + b2_ref[...]


def _head(h, params):
    B, N, C = h.shape
    full = lambda shape: pl.BlockSpec(shape, lambda b: (0,) * len(shape))
    return pl.pallas_call(
        _head_kernel,
        grid=(B,),
        in_specs=[
            pl.BlockSpec((1, N, C), lambda b: (b, 0, 0)),
            full((256, 512)),
            full((256,)),
            full((40, 256)),
            full((40,)),
        ],
        out_specs=pl.BlockSpec((1, 1, 40), lambda b: (b, 0, 0)),
        out_shape=jax.ShapeDtypeStruct((B, 1, 40), jnp.float32),
    )(h, params['fc1_w'], params['fc1_b'], params['fc2_w'],
      params['fc2_b']).reshape(B, 40)


@jax.jit
def kernel(x, params):
    h = x
    for i, (_, c_out) in enumerate(_CH):
        h = _xconv_layer(h, params['xconv%d' % i], c_out)
    return _head(h, params)


# speedup vs baseline: 7.1392x; 1.0001x over previous
import functools
import math

import jax
import jax.numpy as jnp
from jax import lax
from jax.experimental import pallas as pl
from jax.experimental.pallas import tpu as pltpu

_EPS = 1e-5
_K = 8
_CH = [(3, 32), (32, 64), (64, 128), (128, 256), (256, 512)]
_BNS = float(1.0 / math.sqrt(1.0 + _EPS))


def _softmax_rows(a):
    m = jnp.max(a, axis=1, keepdims=True)
    e = jnp.exp(a - m)
    return e / jnp.sum(e, axis=1, keepdims=True)


def _layer_kernel(xb_ref, q_ref, wxt_ref, bxt_ref, gxt_ref, bexr_ref,
                  wc_ref, bc_ref, gc_ref, bec_ref, out_ref):
    xb = xb_ref[0]
    q = q_ref[0]
    R = q.shape[0]
    N = xb.shape[0]

    qn = jnp.sum(q * q, axis=1, keepdims=True)
    xn = jnp.sum(xb * xb, axis=1)[None, :]
    d = -2.0 * lax.dot_general(
        q, xb, (((1,), (1,)), ((), ())),
        preferred_element_type=jnp.float32)
    d = d + qn
    d = d + xn

    iota = lax.broadcasted_iota(jnp.int32, (R, N), 1)
    x0 = xb[:, 0][None, :]

    xg = []
    xtcols = []
    for _ in range(_K):
        m = jnp.min(d, axis=1, keepdims=True)
        ismin = d == m
        sel = jnp.min(jnp.where(ismin, iota, N), axis=1, keepdims=True)
        oh = iota == sel
        ohf = oh.astype(jnp.float32)
        xtcols.append(jnp.sum(ohf * x0, axis=1, keepdims=True))
        xg.append(lax.dot_general(ohf, xb, (((1,), (0,)), ((), ())),
                                  preferred_element_type=jnp.float32))
        d = jnp.where(oh, jnp.float32(1e30), d)

    xt = jnp.concatenate(xtcols, axis=1)
    X = lax.dot_general(xt, wxt_ref[...], (((1,), (1,)), ((), ())),
                        preferred_element_type=jnp.float32)
    X = X + bxt_ref[...][None, :]
    X = X * _BNS * gxt_ref[...][None, :] + bexr_ref[...][None, :]

    g = None
    for i in range(_K):
        s = _softmax_rows(X[:, i * _K:(i + 1) * _K])
        gt = s[:, 0:1] * xg[0]
        for j in range(1, _K):
            gt = gt + s[:, j:j + 1] * xg[j]
        g = gt if g is None else jnp.maximum(g, gt)

    out = lax.dot_general(g, wc_ref[...], (((1,), (1,)), ((), ())),
                          preferred_element_type=jnp.float32)
    out = out + bc_ref[...][None, :]
    out = out * _BNS * gc_ref[...][None, :] + bec_ref[...][None, :]
    out_ref[0] = jnp.maximum(out, 0.0)


def _xconv_layer(x, p, c_out, block_r=256):
    B, N, C = x.shape
    grid = (B, N // block_r)
    full = lambda shape: pl.BlockSpec(shape, lambda b, r: (0,) * len(shape))
    return pl.pallas_call(
        _layer_kernel,
        grid=grid,
        in_specs=[
            pl.BlockSpec((1, N, C), lambda b, r: (b, 0, 0)),
            pl.BlockSpec((1, block_r, C), lambda b, r: (b, r, 0)),
            full((_K * _K, _K)),
            full((_K * _K,)),
            full((_K * _K,)),
            full((_K * _K,)),
            full((c_out, C)),
            full((c_out,)),
            full((c_out,)),
            full((c_out,)),
        ],
        out_specs=pl.BlockSpec((1, block_r, c_out), lambda b, r: (b, r, 0)),
        out_shape=jax.ShapeDtypeStruct((B, N, c_out), jnp.float32),
    )(x, x, p['Wxt'], p['bxt'], p['gxt'], p['betaxt'],
      p['Wc'], p['bc'], p['gc'], p['betac'])


def _head_kernel(h_ref, w1_ref, b1_ref, w2_ref, b2_ref, out_ref):
    h = h_ref[0]
    m = jnp.max(h, axis=0, keepdims=True)
    f = lax.dot_general(m, w1_ref[...], (((1,), (1,)), ((), ())),
                        preferred_element_type=jnp.float32)
    f = jnp.maximum(f + b1_ref[...][None, :], 0.0)
    o = lax.dot_general(f, w2_ref[...], (((1,), (1,)), ((), ())),
                        preferred_element_type=jnp.float32)
    out_ref[0, 0] = o[0] + b2_ref[...]


def _head(h, params):
    B, N, C = h.shape
    full = lambda shape: pl.BlockSpec(shape, lambda b: (0,) * len(shape))
    return pl.pallas_call(
        _head_kernel,
        grid=(B,),
        in_specs=[
            pl.BlockSpec((1, N, C), lambda b: (b, 0, 0)),
            full((256, 512)),
            full((256,)),
            full((40, 256)),
            full((40,)),
        ],
        out_specs=pl.BlockSpec((1, 1, 40), lambda b: (b, 0, 0)),
        out_shape=jax.ShapeDtypeStruct((B, 1, 40), jnp.float32),
    )(h, params['fc1_w'], params['fc1_b'], params['fc2_w'],
      params['fc2_b']).reshape(B, 40)


@jax.jit
def kernel(x, params):
    h = x
    for i, (_, c_out) in enumerate(_CH):
        h = _xconv_layer(h, params['xconv%d' % i], c_out)
    return _head(h, params)
